# baseline (speedup 1.0000x reference)
import functools

import jax
import jax.numpy as jnp
from jax import lax
from jax.experimental import pallas as pl
from jax.experimental.pallas import tpu as pltpu
from jax.experimental.pallas import tpu_sc as plsc

_DIM = 64
_NC = 2
_NS = 16
_NW = _NC * _NS
_CHUNK = 320
_NBUF = 4


def _make_gather(n_tot: int):
    b_per_w = n_tot // _NW
    n_chunks = b_per_w // _CHUNK
    n_groups = n_chunks // _NBUF
    assert n_chunks % _NBUF == 0 and n_groups >= 2
    mesh = plsc.VectorSubcoreMesh(core_axis_name="c", subcore_axis_name="s")

    scratch = (
        [pltpu.VMEM((b_per_w,), jnp.int32)]
        + [pltpu.VMEM((_CHUNK, _DIM), jnp.float32) for _ in range(_NBUF)]
        + [pltpu.SemaphoreType.DMA for _ in range(2 * _NBUF)]
    )

    @functools.partial(
        pl.kernel,
        mesh=mesh,
        out_type=jax.ShapeDtypeStruct((n_tot, _DIM), jnp.float32),
        scratch_types=scratch,
        compiler_params=pltpu.CompilerParams(use_tc_tiling_on_sc=False),
    )
    def gather_kernel(table_hbm, idx_hbm, out_hbm, idx_all, *bufs):
        rows = bufs[:_NBUF]
        gsem = bufs[_NBUF:2 * _NBUF]
        ssem = bufs[2 * _NBUF:]
        wid = lax.axis_index("s") * _NC + lax.axis_index("c")
        base = wid * b_per_w

        pltpu.sync_copy(idx_hbm.at[pl.ds(base, b_per_w)], idx_all)

        def start_gather(i, b):
            pltpu.async_copy(
                table_hbm.at[idx_all.at[pl.ds(i * _CHUNK, _CHUNK)]],
                rows[b], gsem[b])

        def wait_gather(i, b):
            pltpu.make_async_copy(
                table_hbm.at[idx_all.at[pl.ds(i * _CHUNK, _CHUNK)]],
                rows[b], gsem[b]).wait()

        def start_store(i, b):
            pltpu.async_copy(
                rows[b], out_hbm.at[pl.ds(base + i * _CHUNK, _CHUNK)],
                ssem[b])

        def wait_store(i, b):
            pltpu.make_async_copy(
                rows[b], out_hbm.at[pl.ds(base + i * _CHUNK, _CHUNK)],
                ssem[b]).wait()


        for b in range(_NBUF):
            start_gather(b, b)
            j = b - (_NBUF - 1)
            if j >= 0:
                wait_gather(j, j % _NBUF)
                start_store(j, j % _NBUF)

        def group_body(g, carry):
            for b in range(_NBUF):
                i = g * _NBUF + b
                wait_store(i - _NBUF, b)
                start_gather(i, b)
                j = i - (_NBUF - 1)
                bj = (b + 1) % _NBUF
                wait_gather(j, bj)
                start_store(j, bj)
            return carry

        lax.fori_loop(1, n_groups, group_body, 0)

        last = n_chunks - 1
        for j in range(n_chunks - (_NBUF - 1), n_chunks):
            wait_gather(j, j % _NBUF)
            start_store(j, j % _NBUF)
        for i in range(n_chunks - _NBUF, n_chunks):
            wait_store(i, i % _NBUF)

    return gather_kernel


def kernel(input_ids, table):
    b, l = input_ids.shape
    n_tot = b * l
    flat = input_ids.reshape(n_tot)
    out = _make_gather(n_tot)(table, flat)
    return out.reshape(b, l, _DIM)
